# baseline (speedup 1.0000x reference)
import functools

import jax
import jax.numpy as jnp
from jax import lax
from jax.experimental import pallas as pl
from jax.experimental.pallas import tpu as pltpu

V = 2048
D_MODEL = 512
D_FFN = 1024
NUM_BLOCKS = 2

BV = 256
TR = 256


def _bdot(a, b):
    return jnp.dot(a.astype(jnp.bfloat16), b.astype(jnp.bfloat16),
                   preferred_element_type=jnp.float32)


def _full(shape):
    return pl.BlockSpec(shape, lambda *_: tuple(0 for _ in shape))


def _variant_body(info_ref, patches_ref, wi1, bi1, wi2, bi2, wconv, bconv,
                  wseq, bseq, iseq_out):
    info = info_ref[...]
    e = jnp.maximum(jnp.dot(info, wi1[...], preferred_element_type=jnp.float32)
                    + bi1[...], 0.0)
    e = jnp.maximum(jnp.dot(e, wi2[...], preferred_element_type=jnp.float32)
                    + bi2[...], 0.0)
    p = patches_ref[...]
    c = jnp.dot(p, wconv[...], preferred_element_type=jnp.float32) + bconv[...]
    c = jnp.maximum(c, 0.0)
    c = c.reshape(BV, 60, 64)
    feat = jnp.max(c, axis=1)
    s = jnp.maximum(jnp.dot(feat, wseq[...], preferred_element_type=jnp.float32)
                    + bseq[...], 0.0)
    iseq_out[:, :128] = e
    iseq_out[:, 128:] = s


def _variant_stage(info_2d, patches, W_i1, b_i1, W_i2, b_i2, W_conv2d, b_conv,
                   W_seq, b_seq):
    grid = (V // BV,)
    return pl.pallas_call(
        _variant_body,
        grid=grid,
        in_specs=[
            pl.BlockSpec((BV, 64), lambda i: (i, 0)),
            pl.BlockSpec((BV * 60, 24), lambda i: (i, 0)),
            _full((64, 128)), _full((1, 128)),
            _full((128, 128)), _full((1, 128)),
            _full((24, 64)), _full((1, 64)),
            _full((64, 128)), _full((1, 128)),
        ],
        out_specs=pl.BlockSpec((BV, 256), lambda i: (i, 0)),
        out_shape=jax.ShapeDtypeStruct((V, 256), jnp.float32),
    )(info_2d, patches, W_i1, b_i1, W_i2, b_i2, W_conv2d, b_conv, W_seq, b_seq)


def _read_body(reads_ref, iseq_g_ref, w1, b1, w2, b2, x_out):
    r = reads_ref[...]
    e = jnp.maximum(_bdot(r, w1[...]) + b1[...], 0.0)
    e = jnp.maximum(_bdot(e, w2[...]) + b2[...], 0.0)
    x_out[:, :256] = e
    x_out[:, 256:] = iseq_g_ref[...]


def _read_stage(reads_2d, iseq_g, W_r1, b_r1, W_r2, b_r2, ntiles):
    total = reads_2d.shape[0]
    return pl.pallas_call(
        _read_body,
        grid=(ntiles,),
        in_specs=[
            pl.BlockSpec((TR, 128), lambda i: (i, 0)),
            pl.BlockSpec((TR, 256), lambda i: (i, 0)),
            _full((128, 256)), _full((1, 256)),
            _full((256, 256)), _full((1, 256)),
        ],
        out_specs=pl.BlockSpec((TR, D_MODEL), lambda i: (i, 0)),
        out_shape=jax.ShapeDtypeStruct((total, D_MODEL), jnp.float32),
    )(reads_2d, iseq_g, W_r1, b_r1, W_r2, b_r2)


def _ctx_body(sr_ref, sa_ref, rc_ref, ac_ref, wc0, wc1, ctx0_out, ctx1_out):
    ref_mean = sr_ref[...] / rc_ref[...]
    alt_mean = sa_ref[...] / ac_ref[...]
    ctx0_out[...] = _bdot(alt_mean, wc0[...])
    ctx1_out[...] = _bdot(ref_mean, wc1[...])


def _ctx_stage(sum_ref, sum_alt, rc, ac, Wc0, Wc1):
    grid = (V // BV,)
    spec = pl.BlockSpec((BV, D_MODEL), lambda i: (i, 0))
    ospec = pl.BlockSpec((BV, D_FFN), lambda i: (i, 0))
    return pl.pallas_call(
        _ctx_body,
        grid=grid,
        in_specs=[spec, spec,
                  pl.BlockSpec((BV, 1), lambda i: (i, 0)),
                  pl.BlockSpec((BV, 1), lambda i: (i, 0)),
                  _full((D_MODEL, D_FFN)), _full((D_MODEL, D_FFN))],
        out_specs=[ospec, ospec],
        out_shape=[jax.ShapeDtypeStruct((V, D_FFN), jnp.float32),
                   jax.ShapeDtypeStruct((V, D_FFN), jnp.float32)],
    )(sum_ref, sum_alt, rc, ac, Wc0, Wc1)


def _enc_body(tix_s, wsel_s, init_s, r_s, x_ref, ctx_ref, w1, w2, w3, x_out):
    g = pl.program_id(0)
    rows = tix_s[g] * TR + lax.broadcasted_iota(jnp.int32, (TR, 1), 0)
    m = jnp.logical_xor(rows < r_s[0], wsel_s[g] == 1)
    x = x_ref[...]
    xb = x.astype(jnp.bfloat16)
    u = jnp.dot(xb, w1[0].astype(jnp.bfloat16),
                preferred_element_type=jnp.float32)
    gg = jax.nn.sigmoid(jnp.dot(xb, w2[0].astype(jnp.bfloat16),
                                preferred_element_type=jnp.float32)
                        + ctx_ref[...])
    d = _bdot(u * gg, w3[0])
    d = jnp.where(m, d, 0.0)

    @pl.when(init_s[g] == 1)
    def _():
        x_out[...] = x + d

    @pl.when(init_s[g] == 0)
    def _():
        x_out[...] = x_out[...] + d


def _enc_stage(tix, wsel, init, Rarr, x, ctx_g, W1b, W2b, W3b, nsteps):
    total = x.shape[0]
    rd = lambda g, tix, wsel, init, r: (tix[g], 0)
    wt = lambda g, tix, wsel, init, r: (wsel[g], 0, 0)
    grid_spec = pltpu.PrefetchScalarGridSpec(
        num_scalar_prefetch=4,
        grid=(nsteps,),
        in_specs=[
            pl.BlockSpec((TR, D_MODEL), rd),
            pl.BlockSpec((TR, D_FFN), rd),
            pl.BlockSpec((1, D_MODEL, D_FFN), wt),
            pl.BlockSpec((1, D_MODEL, D_FFN), wt),
            pl.BlockSpec((1, D_FFN, D_MODEL), wt),
        ],
        out_specs=pl.BlockSpec((TR, D_MODEL), rd),
    )
    return pl.pallas_call(
        _enc_body,
        grid_spec=grid_spec,
        out_shape=jax.ShapeDtypeStruct((total, D_MODEL), jnp.float32),
    )(tix, wsel, init, Rarr, x, ctx_g, W1b, W2b, W3b)


def _phi_body(skip_s, x_ref, wp1, bp1, wp2, bp2, phi_out):
    @pl.when(skip_s[pl.program_id(0)] == 0)
    def _():
        x = x_ref[...]
        p = jnp.maximum(_bdot(x, wp1[...]) + bp1[...], 0.0)
        p = jnp.maximum(_bdot(p, wp2[...]) + bp2[...], 0.0)
        phi_out[...] = p


def _phi_stage(skip, x, W_p1, b_p1, W_p2, b_p2, ntiles):
    total = x.shape[0]
    rd = lambda i, s: (i, 0)
    grid_spec = pltpu.PrefetchScalarGridSpec(
        num_scalar_prefetch=1,
        grid=(ntiles,),
        in_specs=[
            pl.BlockSpec((TR, D_MODEL), rd),
            pl.BlockSpec((D_MODEL, 1024), lambda i, s: (0, 0)),
            pl.BlockSpec((1, 1024), lambda i, s: (0, 0)),
            pl.BlockSpec((1024, 1024), lambda i, s: (0, 0)),
            pl.BlockSpec((1, 1024), lambda i, s: (0, 0)),
        ],
        out_specs=pl.BlockSpec((TR, 1024), rd),
    )
    return pl.pallas_call(
        _phi_body,
        grid_spec=grid_spec,
        out_shape=jax.ShapeDtypeStruct((total, 1024), jnp.float32),
    )(skip, x, W_p1, b_p1, W_p2, b_p2)


def _final_body(pooled_ref, ac_ref, wf1, bf1, wf2, bf2, out_ref):
    pooled = pooled_ref[...] / ac_ref[...]
    h = jnp.maximum(_bdot(pooled, wf1[...]) + bf1[...], 0.0)
    out_ref[...] = _bdot(h, wf2[...]) + bf2[...]


def _final_stage(pool_sum, ac, W_f1, b_f1, W_f2, b_f2):
    grid = (V // BV,)
    return pl.pallas_call(
        _final_body,
        grid=grid,
        in_specs=[
            pl.BlockSpec((BV, 1024), lambda i: (i, 0)),
            pl.BlockSpec((BV, 1), lambda i: (i, 0)),
            _full((1024, 512)), _full((1, 512)),
            _full((512, 256)), _full((1, 256)),
        ],
        out_specs=pl.BlockSpec((BV, 256), lambda i: (i, 0)),
        out_shape=jax.ShapeDtypeStruct((V, 256), jnp.float32),
    )(pool_sum, ac, W_f1, b_f1, W_f2, b_f2)


def kernel(reads_2d, info_2d, ref_seq_2d, W_r1, b_r1, W_r2, b_r2, W_i1, b_i1,
           W_i2, b_i2, W_conv, b_conv, W_seq, b_seq, enc_W1, enc_W2, enc_Wc,
           enc_W3, W_p1, b_p1, W_p2, b_p2, W_f1, b_f1, W_f2, b_f2,
           ref_counts, alt_counts):
    total = reads_2d.shape[0]
    ntiles = (total + TR - 1) // TR

    counts2 = jnp.concatenate((ref_counts, alt_counts)).astype(jnp.int32)
    seg2 = jnp.repeat(jnp.arange(2 * V, dtype=jnp.int32), counts2,
                      total_repeat_length=total)
    is_alt = seg2 >= V
    var_all = jnp.where(is_alt, seg2 - V, seg2)
    R = jnp.sum(ref_counts).astype(jnp.int32)

    rc = ref_counts.astype(jnp.float32).reshape(V, 1)
    ac = alt_counts.astype(jnp.float32).reshape(V, 1)

    x3 = ref_seq_2d.reshape(V, 4, 64)
    cols = [x3[:, i, k:k + 60] for i in range(4) for k in range(5)]
    patches = jnp.stack(cols + [jnp.zeros((V, 60), jnp.float32)] * 4, axis=-1)
    patches = patches.reshape(V * 60, 24)
    W_conv2d = jnp.concatenate(
        [W_conv.reshape(64, 20).T, jnp.zeros((4, 64), jnp.float32)], axis=0)

    iseq = _variant_stage(info_2d, patches, W_i1, b_i1.reshape(1, -1),
                          W_i2, b_i2.reshape(1, -1), W_conv2d,
                          b_conv.reshape(1, -1), W_seq, b_seq.reshape(1, -1))
    ref_seq_embeddings_ve = iseq[:, 128:]

    iseq_g = jnp.take(iseq, var_all, axis=0)
    x = _read_stage(reads_2d, iseq_g, W_r1, b_r1.reshape(1, -1),
                    W_r2, b_r2.reshape(1, -1), ntiles)

    nsteps = ntiles + 1
    s_t = R // TR
    gidx = jnp.arange(nsteps, dtype=jnp.int32)
    tix = jnp.where(gidx <= s_t, gidx, gidx - 1).astype(jnp.int32)
    wsel = (gidx > s_t).astype(jnp.int32)
    init = jnp.where(gidx == s_t + 1, 0, 1).astype(jnp.int32)
    Rarr = R.reshape(1)
    skip = ((jnp.arange(ntiles, dtype=jnp.int32) + 1) * TR <= R).astype(jnp.int32)

    zero = jnp.zeros((), jnp.float32)
    alt_col = is_alt[:, None]
    for b in range(NUM_BLOCKS):
        sums = jax.ops.segment_sum(x, seg2, num_segments=2 * V)
        ctx0, ctx1 = _ctx_stage(sums[:V], sums[V:], rc, ac,
                                enc_Wc[b, 0], enc_Wc[b, 1])
        ctx_g = jnp.where(alt_col, jnp.take(ctx1, var_all, axis=0),
                          jnp.take(ctx0, var_all, axis=0))
        x = _enc_stage(tix, wsel, init, Rarr, x, ctx_g,
                       enc_W1[b], enc_W2[b], enc_W3[b], nsteps)

    phi = _phi_stage(skip, x, W_p1, b_p1.reshape(1, -1), W_p2,
                     b_p2.reshape(1, -1), ntiles)
    phi_alt = jnp.where(alt_col, phi, zero)
    pool_sum = jax.ops.segment_sum(phi_alt, var_all, num_segments=V)
    result_be = _final_stage(pool_sum, ac, W_f1, b_f1.reshape(1, -1),
                             W_f2, b_f2.reshape(1, -1))
    return result_be, ref_seq_embeddings_ve


# speedup vs baseline: 2.8510x; 2.8510x over previous
import functools

import jax
import jax.numpy as jnp
from jax import lax
from jax.experimental import pallas as pl
from jax.experimental.pallas import tpu as pltpu

V = 2048
D_MODEL = 512
D_FFN = 1024
NUM_BLOCKS = 2

BV = 256
TR = 256
W = 256
BIG = 10 * V


def _bf(x):
    return x.astype(jnp.bfloat16)


def _bdot(a, b):
    return jnp.dot(_bf(a), _bf(b), preferred_element_type=jnp.float32)


def _full(shape):
    return pl.BlockSpec(shape, lambda *_: tuple(0 for _ in shape))


def _fullp(shape):
    return pl.BlockSpec(shape, lambda g, *s: tuple(0 for _ in shape))


def _variant_body(info_ref, patches_ref, wi1, bi1, wi2, bi2, wconv, bconv,
                  wseq, bseq, iseq_out):
    info = info_ref[...]
    e = jnp.maximum(jnp.dot(info, wi1[...], preferred_element_type=jnp.float32)
                    + bi1[...], 0.0)
    e = jnp.maximum(jnp.dot(e, wi2[...], preferred_element_type=jnp.float32)
                    + bi2[...], 0.0)
    p = patches_ref[...]
    c = jnp.dot(p, wconv[...], preferred_element_type=jnp.float32) + bconv[...]
    c = jnp.maximum(c, 0.0)
    c = c.reshape(BV, 60, 64)
    feat = jnp.max(c, axis=1)
    s = jnp.maximum(jnp.dot(feat, wseq[...], preferred_element_type=jnp.float32)
                    + bseq[...], 0.0)
    iseq_out[:, :128] = e
    iseq_out[:, 128:] = s


def _variant_stage(info_2d, patches, W_i1, b_i1, W_i2, b_i2, W_conv2d, b_conv,
                   W_seq, b_seq):
    return pl.pallas_call(
        _variant_body,
        grid=(V // BV,),
        in_specs=[
            pl.BlockSpec((BV, 64), lambda i: (i, 0)),
            pl.BlockSpec((BV * 60, 24), lambda i: (i, 0)),
            _full((64, 128)), _full((1, 128)),
            _full((128, 128)), _full((1, 128)),
            _full((24, 64)), _full((1, 64)),
            _full((64, 128)), _full((1, 128)),
        ],
        out_specs=pl.BlockSpec((BV, 256), lambda i: (i, 0)),
        out_shape=jax.ShapeDtypeStruct((V, 256), jnp.float32),
    )(info_2d, patches, W_i1, b_i1, W_i2, b_i2, W_conv2d, b_conv, W_seq, b_seq)


def _read_body(wb_s, wbase_s, ir_s, reads_ref, segr_ref, w1, b1, w2, b2,
               xa_out, o1_out, o2_out, *, total):
    g = pl.program_id(0)
    r = reads_ref[...]
    e = jnp.maximum(_bdot(r, w1[...]) + b1[...], 0.0)
    e = jnp.maximum(_bdot(e, w2[...]) + b2[...], 0.0)
    rows = g * TR + lax.broadcasted_iota(jnp.int32, (TR, 1), 0)
    e = jnp.where(rows < total, e, 0.0)
    xa_out[...] = e

    segr = segr_ref[0]
    base = wbase_s[g]
    ior = lax.broadcasted_iota(jnp.int32, (W, TR), 0)
    ohT1 = _bf(ior == (segr - base))
    ohT2 = _bf(ior == (segr - base - W))
    eb = _bf(e)
    p1 = jnp.dot(ohT1, eb, preferred_element_type=jnp.float32)
    p2 = jnp.dot(ohT2, eb, preferred_element_type=jnp.float32)

    @pl.when(ir_s[g] == 1)
    def _():
        o1_out[...] = p1
        o2_out[...] = p2

    @pl.when(ir_s[g] == 0)
    def _():
        o1_out[...] = o1_out[...] + p1
        o2_out[...] = o2_out[...] + p2


def _read_stage(wb, wbase, ir, reads_2d, seg_row, W_r1, b_r1, W_r2, b_r2,
                ntiles, total):
    body = functools.partial(_read_body, total=total)
    grid_spec = pltpu.PrefetchScalarGridSpec(
        num_scalar_prefetch=3,
        grid=(ntiles,),
        in_specs=[
            pl.BlockSpec((TR, 128), lambda g, *s: (g, 0)),
            pl.BlockSpec((1, 1, TR), lambda g, *s: (g, 0, 0)),
            _fullp((128, 256)), _fullp((1, 256)),
            _fullp((256, 256)), _fullp((1, 256)),
        ],
        out_specs=[
            pl.BlockSpec((TR, 256), lambda g, *s: (g, 0)),
            pl.BlockSpec((W, 256), lambda g, wb_s, *s: (wb_s[g], 0)),
            pl.BlockSpec((W, 256), lambda g, wb_s, *s: (wb_s[g] + 1, 0)),
        ],
    )
    return pl.pallas_call(
        body,
        grid_spec=grid_spec,
        out_shape=[jax.ShapeDtypeStruct((ntiles * TR, 256), jnp.float32),
                   jax.ShapeDtypeStruct((2 * V + W, 256), jnp.float32),
                   jax.ShapeDtypeStruct((2 * V + W, 256), jnp.float32)],
    )(wb, wbase, ir, reads_2d, seg_row, W_r1, b_r1, W_r2, b_r2)


def _ctx_body(sr_ref, sa_ref, rc_ref, ac_ref, wc0, wc1, ctx0_out, ctx1_out):
    ref_mean = sr_ref[...] / rc_ref[...]
    alt_mean = sa_ref[...] / ac_ref[...]
    ctx0_out[...] = _bdot(alt_mean, wc0[...])
    ctx1_out[...] = _bdot(ref_mean, wc1[...])


def _ctx_stage(sum_ref, sum_alt, rc, ac, Wc0, Wc1):
    spec = pl.BlockSpec((BV, D_MODEL), lambda i: (i, 0))
    ospec = pl.BlockSpec((BV, D_FFN), lambda i: (i, 0))
    return pl.pallas_call(
        _ctx_body,
        grid=(V // BV,),
        in_specs=[spec, spec,
                  pl.BlockSpec((BV, 1), lambda i: (i, 0)),
                  pl.BlockSpec((BV, 1), lambda i: (i, 0)),
                  _full((D_MODEL, D_FFN)), _full((D_MODEL, D_FFN))],
        out_specs=[ospec, ospec],
        out_shape=[jax.ShapeDtypeStruct((V, D_FFN), jnp.float32),
                   jax.ShapeDtypeStruct((V, D_FFN), jnp.float32)],
    )(sum_ref, sum_alt, rc, ac, Wc0, Wc1)


def _enc0_body(tix_s, wsel_s, init_s, r_s, wb_s, wbase_s, sinit_s,
               xa_ref, xb_ref, segc_ref, segr_ref, cw1_ref, cw2_ref,
               w1, w2, w3, x_out, d1_out, d2_out, *, total):
    g = pl.program_id(0)
    rows = tix_s[g] * TR + lax.broadcasted_iota(jnp.int32, (TR, 1), 0)
    m = jnp.logical_and(jnp.logical_xor(rows < r_s[0], wsel_s[g] == 1),
                        rows < total)
    base = wbase_s[g]
    segc = segc_ref[...]
    ioc = lax.broadcasted_iota(jnp.int32, (TR, W), 1)
    oh1 = _bf((segc - base) == ioc)
    oh2 = _bf((segc - base - W) == ioc)
    gctx = jnp.dot(oh1, _bf(cw1_ref[...]), preferred_element_type=jnp.float32) \
        + jnp.dot(oh2, _bf(cw2_ref[...]), preferred_element_type=jnp.float32)

    xb16 = jnp.concatenate([_bf(xa_ref[...]), _bf(xb_ref[...])], axis=1)
    u = jnp.dot(xb16, _bf(w1[0]), preferred_element_type=jnp.float32)
    gg = jax.nn.sigmoid(
        jnp.dot(xb16, _bf(w2[0]), preferred_element_type=jnp.float32) + gctx)
    d = _bdot(u * gg, w3[0])
    d = jnp.where(m, d, 0.0)

    segr = segr_ref[0]
    ior = lax.broadcasted_iota(jnp.int32, (W, TR), 0)
    ohT1 = _bf(ior == (segr - base))
    ohT2 = _bf(ior == (segr - base - W))
    db = _bf(d)
    p1 = jnp.dot(ohT1, db, preferred_element_type=jnp.float32)
    p2 = jnp.dot(ohT2, db, preferred_element_type=jnp.float32)

    @pl.when(init_s[g] == 1)
    def _():
        x = jnp.concatenate([xa_ref[...], xb_ref[...]], axis=1)
        x_out[...] = x + d

    @pl.when(init_s[g] == 0)
    def _():
        x_out[...] = x_out[...] + d

    @pl.when(sinit_s[g] == 1)
    def _():
        d1_out[...] = p1
        d2_out[...] = p2

    @pl.when(sinit_s[g] == 0)
    def _():
        d1_out[...] = d1_out[...] + p1
        d2_out[...] = d2_out[...] + p2


def _enc0_stage(scalars, xa, xbg, seg_col, seg_row, ctx_tab, W1b, W2b, W3b,
                nsteps, total):
    body = functools.partial(_enc0_body, total=total)
    rd = lambda g, tix, *s: (tix[g], 0)
    rd3 = lambda g, tix, *s: (tix[g], 0, 0)
    wt = lambda g, tix, wsel, *s: (wsel[g], 0, 0)
    cw1 = lambda g, tix, wsel, init, r, wb, *s: (wb[g], 0)
    cw2 = lambda g, tix, wsel, init, r, wb, *s: (wb[g] + 1, 0)
    grid_spec = pltpu.PrefetchScalarGridSpec(
        num_scalar_prefetch=7,
        grid=(nsteps,),
        in_specs=[
            pl.BlockSpec((TR, 256), rd),
            pl.BlockSpec((TR, 256), rd),
            pl.BlockSpec((TR, 1), rd),
            pl.BlockSpec((1, 1, TR), rd3),
            pl.BlockSpec((W, D_FFN), cw1),
            pl.BlockSpec((W, D_FFN), cw2),
            pl.BlockSpec((1, D_MODEL, D_FFN), wt),
            pl.BlockSpec((1, D_MODEL, D_FFN), wt),
            pl.BlockSpec((1, D_FFN, D_MODEL), wt),
        ],
        out_specs=[
            pl.BlockSpec((TR, D_MODEL), rd),
            pl.BlockSpec((W, D_MODEL), cw1),
            pl.BlockSpec((W, D_MODEL), cw2),
        ],
    )
    nrows = xa.shape[0]
    return pl.pallas_call(
        body,
        grid_spec=grid_spec,
        out_shape=[jax.ShapeDtypeStruct((nrows, D_MODEL), jnp.float32),
                   jax.ShapeDtypeStruct((2 * V + W, D_MODEL), jnp.float32),
                   jax.ShapeDtypeStruct((2 * V + W, D_MODEL), jnp.float32)],
    )(*scalars, xa, xbg, seg_col, seg_row, ctx_tab, ctx_tab, W1b, W2b, W3b)


def _enc1_body(tix_s, wsel_s, init_s, r_s, wb_s, wbase_s, pwb_s, pwbase_s,
               pinit_s, x_ref, segc_ref, segr_ref, cw1_ref, cw2_ref,
               w1, w2, w3, wp1, bp1, wp2, bp2, p1_out, p2_out, *, total):
    g = pl.program_id(0)
    do_phi = wsel_s[g] == 1

    @pl.when(jnp.logical_and(pinit_s[g] == 1, jnp.logical_not(do_phi)))
    def _():
        p1_out[...] = jnp.zeros_like(p1_out)
        p2_out[...] = jnp.zeros_like(p2_out)

    @pl.when(do_phi)
    def _():
        rows = tix_s[g] * TR + lax.broadcasted_iota(jnp.int32, (TR, 1), 0)
        m = jnp.logical_and(rows >= r_s[0], rows < total)
        base = wbase_s[g]
        segc = segc_ref[...]
        ioc = lax.broadcasted_iota(jnp.int32, (TR, W), 1)
        oh1 = _bf((segc - base) == ioc)
        oh2 = _bf((segc - base - W) == ioc)
        gctx = jnp.dot(oh1, _bf(cw1_ref[...]),
                       preferred_element_type=jnp.float32) \
            + jnp.dot(oh2, _bf(cw2_ref[...]),
                      preferred_element_type=jnp.float32)

        x = x_ref[...]
        xb16 = _bf(x)
        u = jnp.dot(xb16, _bf(w1[0]), preferred_element_type=jnp.float32)
        gg = jax.nn.sigmoid(
            jnp.dot(xb16, _bf(w2[0]), preferred_element_type=jnp.float32)
            + gctx)
        d = _bdot(u * gg, w3[0])
        d = jnp.where(m, d, 0.0)
        xn = jnp.where(m, x + d, 0.0)
        ph = jnp.maximum(_bdot(xn, wp1[...]) + bp1[...], 0.0)
        ph = jnp.maximum(_bdot(ph, wp2[...]) + bp2[...], 0.0)
        pbase = pwbase_s[g]
        segr = segr_ref[0]
        ior = lax.broadcasted_iota(jnp.int32, (W, TR), 0)
        ohp1 = _bf(ior == (segr - pbase))
        ohp2 = _bf(ior == (segr - pbase - W))
        phb = _bf(ph)
        q1 = jnp.dot(ohp1, phb, preferred_element_type=jnp.float32)
        q2 = jnp.dot(ohp2, phb, preferred_element_type=jnp.float32)

        @pl.when(pinit_s[g] == 1)
        def _():
            p1_out[...] = q1
            p2_out[...] = q2

        @pl.when(pinit_s[g] == 0)
        def _():
            p1_out[...] = p1_out[...] + q1
            p2_out[...] = p2_out[...] + q2


def _enc1_stage(scalars, x, seg_col, seg_row, ctx_tab, W1b, W2b, W3b,
                W_p1, b_p1, W_p2, b_p2, nsteps, total):
    body = functools.partial(_enc1_body, total=total)
    rd = lambda g, tix, *s: (tix[g], 0)
    rd3 = lambda g, tix, *s: (tix[g], 0, 0)
    wt = lambda g, tix, wsel, *s: (wsel[g], 0, 0)
    cw1 = lambda g, tix, wsel, init, r, wb, *s: (wb[g], 0)
    cw2 = lambda g, tix, wsel, init, r, wb, *s: (wb[g] + 1, 0)
    pw1 = lambda g, tix, wsel, init, r, wb, wbase, pwb, *s: (pwb[g], 0)
    pw2 = lambda g, tix, wsel, init, r, wb, wbase, pwb, *s: (pwb[g] + 1, 0)
    grid_spec = pltpu.PrefetchScalarGridSpec(
        num_scalar_prefetch=9,
        grid=(nsteps,),
        in_specs=[
            pl.BlockSpec((TR, D_MODEL), rd),
            pl.BlockSpec((TR, 1), rd),
            pl.BlockSpec((1, 1, TR), rd3),
            pl.BlockSpec((W, D_FFN), cw1),
            pl.BlockSpec((W, D_FFN), cw2),
            pl.BlockSpec((1, D_MODEL, D_FFN), wt),
            pl.BlockSpec((1, D_MODEL, D_FFN), wt),
            pl.BlockSpec((1, D_FFN, D_MODEL), wt),
            _fullp((D_MODEL, 1024)), _fullp((1, 1024)),
            _fullp((1024, 1024)), _fullp((1, 1024)),
        ],
        out_specs=[
            pl.BlockSpec((W, 1024), pw1),
            pl.BlockSpec((W, 1024), pw2),
        ],
    )
    return pl.pallas_call(
        body,
        grid_spec=grid_spec,
        out_shape=[jax.ShapeDtypeStruct((V + W, 1024), jnp.float32),
                   jax.ShapeDtypeStruct((V + W, 1024), jnp.float32)],
    )(*scalars, x, seg_col, seg_row, ctx_tab, ctx_tab, W1b, W2b, W3b,
      W_p1, b_p1, W_p2, b_p2)


def _final_body(pooled_ref, ac_ref, wf1, bf1, wf2, bf2, out_ref):
    pooled = pooled_ref[...] / ac_ref[...]
    h = jnp.maximum(_bdot(pooled, wf1[...]) + bf1[...], 0.0)
    out_ref[...] = _bdot(h, wf2[...]) + bf2[...]


def _final_stage(pool_sum, ac, W_f1, b_f1, W_f2, b_f2):
    return pl.pallas_call(
        _final_body,
        grid=(V // BV,),
        in_specs=[
            pl.BlockSpec((BV, 1024), lambda i: (i, 0)),
            pl.BlockSpec((BV, 1), lambda i: (i, 0)),
            _full((1024, 512)), _full((1, 512)),
            _full((512, 256)), _full((1, 256)),
        ],
        out_specs=pl.BlockSpec((BV, 256), lambda i: (i, 0)),
        out_shape=jax.ShapeDtypeStruct((V, 256), jnp.float32),
    )(pool_sum, ac, W_f1, b_f1, W_f2, b_f2)


def _merge_windows(o1, o2, n):
    return o1[:n] + jnp.concatenate(
        [jnp.zeros((W, o2.shape[1]), o2.dtype), o2[W:n]], axis=0)


def kernel(reads_2d, info_2d, ref_seq_2d, W_r1, b_r1, W_r2, b_r2, W_i1, b_i1,
           W_i2, b_i2, W_conv, b_conv, W_seq, b_seq, enc_W1, enc_W2, enc_Wc,
           enc_W3, W_p1, b_p1, W_p2, b_p2, W_f1, b_f1, W_f2, b_f2,
           ref_counts, alt_counts):
    total = reads_2d.shape[0]
    ntiles = (total + TR - 1) // TR

    counts2 = jnp.concatenate((ref_counts, alt_counts)).astype(jnp.int32)
    seg2 = jnp.repeat(jnp.arange(2 * V, dtype=jnp.int32), counts2,
                      total_repeat_length=total)
    is_alt = seg2 >= V
    var_all = jnp.where(is_alt, seg2 - V, seg2)
    R = jnp.sum(ref_counts).astype(jnp.int32)

    rc = ref_counts.astype(jnp.float32).reshape(V, 1)
    ac = alt_counts.astype(jnp.float32).reshape(V, 1)

    seg_pad = jnp.concatenate(
        [seg2, jnp.full((ntiles * TR - total,), BIG, jnp.int32)])
    seg_col = seg_pad.reshape(ntiles * TR, 1)
    seg_row = seg_pad.reshape(ntiles, 1, TR)

    seg_start = seg2[::TR]
    wb_t = (seg_start // W).astype(jnp.int32)
    va_t = jnp.maximum(seg_start - V, 0)
    pwb_t = (va_t // W).astype(jnp.int32)

    ir = jnp.concatenate([jnp.ones((1,), jnp.int32),
                          (wb_t[1:] != wb_t[:-1]).astype(jnp.int32)])

    nsteps = ntiles + 1
    s_t = R // TR
    gidx = jnp.arange(nsteps, dtype=jnp.int32)
    tix = jnp.where(gidx <= s_t, gidx, gidx - 1).astype(jnp.int32)
    wsel = (gidx > s_t).astype(jnp.int32)
    init = jnp.where(gidx == s_t + 1, 0, 1).astype(jnp.int32)
    Rarr = jnp.stack([R, jnp.full((), V, jnp.int32)]).astype(jnp.int32)
    wb_g = wb_t[tix]
    wbase_g = wb_g * W
    sinit = jnp.concatenate([jnp.ones((1,), jnp.int32),
                             (wb_g[1:] != wb_g[:-1]).astype(jnp.int32)])
    pwb_g = pwb_t[tix]
    pwbase_g = pwb_g * W + V
    pinit = jnp.concatenate([jnp.ones((1,), jnp.int32),
                             (pwb_g[1:] != pwb_g[:-1]).astype(jnp.int32)])

    x3 = ref_seq_2d.reshape(V, 4, 64)
    cols = [x3[:, i, k:k + 60] for i in range(4) for k in range(5)]
    patches = jnp.stack(cols + [jnp.zeros((V, 60), jnp.float32)] * 4, axis=-1)
    patches = patches.reshape(V * 60, 24)
    W_conv2d = jnp.concatenate(
        [W_conv.reshape(64, 20).T, jnp.zeros((4, 64), jnp.float32)], axis=0)

    iseq = _variant_stage(info_2d, patches, W_i1, b_i1.reshape(1, -1),
                          W_i2, b_i2.reshape(1, -1), W_conv2d,
                          b_conv.reshape(1, -1), W_seq, b_seq.reshape(1, -1))
    ref_seq_embeddings_ve = iseq[:, 128:]

    xa, o1, o2 = _read_stage(wb_t, wb_t * W, ir, reads_2d, seg_row,
                             W_r1, b_r1.reshape(1, -1), W_r2,
                             b_r2.reshape(1, -1), ntiles, total)
    sum_a = _merge_windows(o1, o2, 2 * V)

    iseq_g = jnp.take(iseq, var_all, axis=0)
    iseq_g = jnp.concatenate(
        [iseq_g, jnp.zeros((ntiles * TR - total, 256), jnp.float32)])

    cnt_iseq = jnp.tile(iseq, (2, 1)) * counts2[:, None].astype(jnp.float32)
    sums = jnp.concatenate([sum_a, cnt_iseq], axis=1)

    zpad = jnp.zeros((W, 1024), jnp.float32)
    ctx0, ctx1 = _ctx_stage(sums[:V], sums[V:], rc, ac,
                            enc_Wc[0, 0], enc_Wc[0, 1])
    ctx_tab = jnp.concatenate([ctx0, ctx1, zpad], axis=0)

    scalars0 = (tix, wsel, init, Rarr, wb_g, wbase_g, sinit)
    x1, d1, d2 = _enc0_stage(scalars0, xa, iseq_g, seg_col, seg_row, ctx_tab,
                             enc_W1[0], enc_W2[0], enc_W3[0], nsteps, total)
    sums1 = sums + _merge_windows(d1, d2, 2 * V)

    ctx0b, ctx1b = _ctx_stage(sums1[:V], sums1[V:], rc, ac,
                              enc_Wc[1, 0], enc_Wc[1, 1])
    ctx_tab1 = jnp.concatenate([ctx0b, ctx1b, zpad], axis=0)

    scalars1 = (tix, wsel, init, Rarr, wb_g, wbase_g, pwb_g, pwbase_g, pinit)
    p1, p2 = _enc1_stage(scalars1, x1, seg_col, seg_row, ctx_tab1,
                         enc_W1[1], enc_W2[1], enc_W3[1],
                         W_p1, b_p1.reshape(1, -1), W_p2, b_p2.reshape(1, -1),
                         nsteps, total)
    pool_sum = _merge_windows(p1, p2, V)

    result_be = _final_stage(pool_sum, ac, W_f1, b_f1.reshape(1, -1),
                             W_f2, b_f2.reshape(1, -1))
    return result_be, ref_seq_embeddings_ve


# speedup vs baseline: 2.9775x; 1.0443x over previous
import functools

import jax
import jax.numpy as jnp
from jax import lax
from jax.experimental import pallas as pl
from jax.experimental.pallas import tpu as pltpu

V = 2048
D_MODEL = 512
D_FFN = 1024
NUM_BLOCKS = 2

BV = 256
TR = 256
W = 256
BIG = 10 * V


def _bf(x):
    return x.astype(jnp.bfloat16)


def _bdot(a, b):
    return jnp.dot(_bf(a), _bf(b), preferred_element_type=jnp.float32)


def _full(shape):
    return pl.BlockSpec(shape, lambda *_: tuple(0 for _ in shape))


def _fullp(shape):
    return pl.BlockSpec(shape, lambda g, *s: tuple(0 for _ in shape))


def _variant_body(info_ref, patches_ref, wi1, bi1, wi2, bi2, wconv, bconv,
                  wseq, bseq, iseq_out):
    info = info_ref[...]
    e = jnp.maximum(jnp.dot(info, wi1[...], preferred_element_type=jnp.float32)
                    + bi1[...], 0.0)
    e = jnp.maximum(jnp.dot(e, wi2[...], preferred_element_type=jnp.float32)
                    + bi2[...], 0.0)
    p = patches_ref[...]
    c = jnp.dot(p, wconv[...], preferred_element_type=jnp.float32) + bconv[...]
    c = jnp.maximum(c, 0.0)
    c = c.reshape(BV, 60, 64)
    feat = jnp.max(c, axis=1)
    s = jnp.maximum(jnp.dot(feat, wseq[...], preferred_element_type=jnp.float32)
                    + bseq[...], 0.0)
    iseq_out[:, :128] = e
    iseq_out[:, 128:] = s


def _variant_stage(info_2d, patches, W_i1, b_i1, W_i2, b_i2, W_conv2d, b_conv,
                   W_seq, b_seq):
    return pl.pallas_call(
        _variant_body,
        grid=(V // BV,),
        in_specs=[
            pl.BlockSpec((BV, 64), lambda i: (i, 0)),
            pl.BlockSpec((BV * 60, 24), lambda i: (i, 0)),
            _full((64, 128)), _full((1, 128)),
            _full((128, 128)), _full((1, 128)),
            _full((24, 64)), _full((1, 64)),
            _full((64, 128)), _full((1, 128)),
        ],
        out_specs=pl.BlockSpec((BV, 256), lambda i: (i, 0)),
        out_shape=jax.ShapeDtypeStruct((V, 256), jnp.float32),
    )(info_2d, patches, W_i1, b_i1, W_i2, b_i2, W_conv2d, b_conv, W_seq, b_seq)


def _read_body(wb_s, wbase_s, ir_s, reads_ref, segr_ref, w1, b1, w2, b2,
               xa_out, o1_out, o2_out, *, total):
    g = pl.program_id(0)
    r = reads_ref[...]
    e = jnp.maximum(_bdot(r, w1[...]) + b1[...], 0.0)
    e = jnp.maximum(_bdot(e, w2[...]) + b2[...], 0.0)
    rows = g * TR + lax.broadcasted_iota(jnp.int32, (TR, 1), 0)
    e = jnp.where(rows < total, e, 0.0)
    xa_out[...] = e

    segr = segr_ref[0]
    base = wbase_s[g]
    ior = lax.broadcasted_iota(jnp.int32, (W, TR), 0)
    ohT1 = _bf(ior == (segr - base))
    ohT2 = _bf(ior == (segr - base - W))
    eb = _bf(e)
    p1 = jnp.dot(ohT1, eb, preferred_element_type=jnp.float32)
    p2 = jnp.dot(ohT2, eb, preferred_element_type=jnp.float32)

    @pl.when(ir_s[g] == 1)
    def _():
        o1_out[...] = p1
        o2_out[...] = p2

    @pl.when(ir_s[g] == 0)
    def _():
        o1_out[...] = o1_out[...] + p1
        o2_out[...] = o2_out[...] + p2


def _read_stage(wb, wbase, ir, reads_2d, seg_row, W_r1, b_r1, W_r2, b_r2,
                ntiles, total):
    body = functools.partial(_read_body, total=total)
    grid_spec = pltpu.PrefetchScalarGridSpec(
        num_scalar_prefetch=3,
        grid=(ntiles,),
        in_specs=[
            pl.BlockSpec((TR, 128), lambda g, *s: (g, 0)),
            pl.BlockSpec((1, 1, TR), lambda g, *s: (g, 0, 0)),
            _fullp((128, 256)), _fullp((1, 256)),
            _fullp((256, 256)), _fullp((1, 256)),
        ],
        out_specs=[
            pl.BlockSpec((TR, 256), lambda g, *s: (g, 0)),
            pl.BlockSpec((W, 256), lambda g, wb_s, *s: (wb_s[g], 0)),
            pl.BlockSpec((W, 256), lambda g, wb_s, *s: (wb_s[g] + 1, 0)),
        ],
    )
    return pl.pallas_call(
        body,
        grid_spec=grid_spec,
        out_shape=[jax.ShapeDtypeStruct((ntiles * TR, 256), jnp.float32),
                   jax.ShapeDtypeStruct((2 * V + W, 256), jnp.float32),
                   jax.ShapeDtypeStruct((2 * V + W, 256), jnp.float32)],
    )(wb, wbase, ir, reads_2d, seg_row, W_r1, b_r1, W_r2, b_r2)


def _ctx_body(sr_ref, sa_ref, rc_ref, ac_ref, wc0, wc1, ctx0_out, ctx1_out):
    ref_mean = sr_ref[...] / rc_ref[...]
    alt_mean = sa_ref[...] / ac_ref[...]
    ctx0_out[...] = _bdot(alt_mean, wc0[...])
    ctx1_out[...] = _bdot(ref_mean, wc1[...])


def _ctx_stage(sum_ref, sum_alt, rc, ac, Wc0, Wc1):
    spec = pl.BlockSpec((BV, D_MODEL), lambda i: (i, 0))
    ospec = pl.BlockSpec((BV, D_FFN), lambda i: (i, 0))
    return pl.pallas_call(
        _ctx_body,
        grid=(V // BV,),
        in_specs=[spec, spec,
                  pl.BlockSpec((BV, 1), lambda i: (i, 0)),
                  pl.BlockSpec((BV, 1), lambda i: (i, 0)),
                  _full((D_MODEL, D_FFN)), _full((D_MODEL, D_FFN))],
        out_specs=[ospec, ospec],
        out_shape=[jax.ShapeDtypeStruct((V, D_FFN), jnp.float32),
                   jax.ShapeDtypeStruct((V, D_FFN), jnp.float32)],
    )(sum_ref, sum_alt, rc, ac, Wc0, Wc1)


def _enc0_body(tix_s, wsel_s, init_s, r_s, wb_s, wbase_s, sinit_s,
               xa_ref, xb_ref, segc_ref, segr_ref, cw1_ref, cw2_ref,
               w1, w2, w3, x_out, d1_out, d2_out, *, total):
    g = pl.program_id(0)
    rows = tix_s[g] * TR + lax.broadcasted_iota(jnp.int32, (TR, 1), 0)
    m = jnp.logical_and(jnp.logical_xor(rows < r_s[0], wsel_s[g] == 1),
                        rows < total)
    base = wbase_s[g]
    segc = segc_ref[...]
    ioc = lax.broadcasted_iota(jnp.int32, (TR, W), 1)
    oh1 = _bf((segc - base) == ioc)
    oh2 = _bf((segc - base - W) == ioc)
    gctx = jnp.dot(oh1, _bf(cw1_ref[...]), preferred_element_type=jnp.float32) \
        + jnp.dot(oh2, _bf(cw2_ref[...]), preferred_element_type=jnp.float32)

    xb16 = jnp.concatenate([_bf(xa_ref[...]), _bf(xb_ref[...])], axis=1)
    u = jnp.dot(xb16, _bf(w1[0]), preferred_element_type=jnp.float32)
    gg = jax.nn.sigmoid(
        jnp.dot(xb16, _bf(w2[0]), preferred_element_type=jnp.float32) + gctx)
    d = _bdot(u * gg, w3[0])
    d = jnp.where(m, d, 0.0)

    segr = segr_ref[0]
    ior = lax.broadcasted_iota(jnp.int32, (W, TR), 0)
    ohT1 = _bf(ior == (segr - base))
    ohT2 = _bf(ior == (segr - base - W))
    db = _bf(d)
    p1 = jnp.dot(ohT1, db, preferred_element_type=jnp.float32)
    p2 = jnp.dot(ohT2, db, preferred_element_type=jnp.float32)

    @pl.when(init_s[g] == 1)
    def _():
        x = jnp.concatenate([xa_ref[...], xb_ref[...]], axis=1)
        x_out[...] = x + d

    @pl.when(init_s[g] == 0)
    def _():
        x_out[...] = x_out[...] + d

    @pl.when(sinit_s[g] == 1)
    def _():
        d1_out[...] = p1
        d2_out[...] = p2

    @pl.when(sinit_s[g] == 0)
    def _():
        d1_out[...] = d1_out[...] + p1
        d2_out[...] = d2_out[...] + p2


def _enc0_stage(scalars, xa, xbg, seg_col, seg_row, ctx_tab, W1b, W2b, W3b,
                nsteps, total):
    body = functools.partial(_enc0_body, total=total)
    rd = lambda g, tix, *s: (tix[g], 0)
    rd3 = lambda g, tix, *s: (tix[g], 0, 0)
    wt = lambda g, tix, wsel, *s: (wsel[g], 0, 0)
    cw1 = lambda g, tix, wsel, init, r, wb, *s: (wb[g], 0)
    cw2 = lambda g, tix, wsel, init, r, wb, *s: (wb[g] + 1, 0)
    grid_spec = pltpu.PrefetchScalarGridSpec(
        num_scalar_prefetch=7,
        grid=(nsteps,),
        in_specs=[
            pl.BlockSpec((TR, 256), rd),
            pl.BlockSpec((TR, 256), rd),
            pl.BlockSpec((TR, 1), rd),
            pl.BlockSpec((1, 1, TR), rd3),
            pl.BlockSpec((W, D_FFN), cw1),
            pl.BlockSpec((W, D_FFN), cw2),
            pl.BlockSpec((1, D_MODEL, D_FFN), wt),
            pl.BlockSpec((1, D_MODEL, D_FFN), wt),
            pl.BlockSpec((1, D_FFN, D_MODEL), wt),
        ],
        out_specs=[
            pl.BlockSpec((TR, D_MODEL), rd),
            pl.BlockSpec((W, D_MODEL), cw1),
            pl.BlockSpec((W, D_MODEL), cw2),
        ],
    )
    nrows = xa.shape[0]
    return pl.pallas_call(
        body,
        grid_spec=grid_spec,
        out_shape=[jax.ShapeDtypeStruct((nrows, D_MODEL), jnp.float32),
                   jax.ShapeDtypeStruct((2 * V + W, D_MODEL), jnp.float32),
                   jax.ShapeDtypeStruct((2 * V + W, D_MODEL), jnp.float32)],
    )(*scalars, xa, xbg, seg_col, seg_row, ctx_tab, ctx_tab, W1b, W2b, W3b)


def _enc1_body(tix_s, wsel_s, init_s, r_s, wb_s, wbase_s, pwb_s, pwbase_s,
               pinit_s, x_ref, segc_ref, segr_ref, cw1_ref, cw2_ref,
               w1, w2, w3, wp1, bp1, wp2, bp2, p1_out, p2_out, *, total):
    g = pl.program_id(0)
    do_phi = wsel_s[g] == 1

    @pl.when(jnp.logical_and(pinit_s[g] == 1, jnp.logical_not(do_phi)))
    def _():
        p1_out[...] = jnp.zeros_like(p1_out)
        p2_out[...] = jnp.zeros_like(p2_out)

    @pl.when(do_phi)
    def _():
        rows = tix_s[g] * TR + lax.broadcasted_iota(jnp.int32, (TR, 1), 0)
        m = jnp.logical_and(rows >= r_s[0], rows < total)
        base = wbase_s[g]
        segc = segc_ref[...]
        ioc = lax.broadcasted_iota(jnp.int32, (TR, W), 1)
        oh1 = _bf((segc - base) == ioc)
        oh2 = _bf((segc - base - W) == ioc)
        gctx = jnp.dot(oh1, _bf(cw1_ref[...]),
                       preferred_element_type=jnp.float32) \
            + jnp.dot(oh2, _bf(cw2_ref[...]),
                      preferred_element_type=jnp.float32)

        x = x_ref[...]
        xb16 = _bf(x)
        u = jnp.dot(xb16, _bf(w1[0]), preferred_element_type=jnp.float32)
        gg = jax.nn.sigmoid(
            jnp.dot(xb16, _bf(w2[0]), preferred_element_type=jnp.float32)
            + gctx)
        d = _bdot(u * gg, w3[0])
        d = jnp.where(m, d, 0.0)
        xn = jnp.where(m, x + d, 0.0)
        ph = jnp.maximum(_bdot(xn, wp1[...]) + bp1[...], 0.0)
        ph = jnp.maximum(_bdot(ph, wp2[...]) + bp2[...], 0.0)
        pbase = pwbase_s[g]
        segr = segr_ref[0]
        ior = lax.broadcasted_iota(jnp.int32, (W, TR), 0)
        ohp1 = _bf(ior == (segr - pbase))
        ohp2 = _bf(ior == (segr - pbase - W))
        phb = _bf(ph)
        q1 = jnp.dot(ohp1, phb, preferred_element_type=jnp.float32)
        q2 = jnp.dot(ohp2, phb, preferred_element_type=jnp.float32)

        @pl.when(pinit_s[g] == 1)
        def _():
            p1_out[...] = q1
            p2_out[...] = q2

        @pl.when(pinit_s[g] == 0)
        def _():
            p1_out[...] = p1_out[...] + q1
            p2_out[...] = p2_out[...] + q2


def _enc1_stage(scalars, x, seg_col, seg_row, ctx_tab, W1b, W2b, W3b,
                W_p1, b_p1, W_p2, b_p2, nsteps, total):
    body = functools.partial(_enc1_body, total=total)
    rd = lambda g, tix, *s: (tix[g], 0)
    rd3 = lambda g, tix, *s: (tix[g], 0, 0)
    wt = lambda g, tix, wsel, *s: (wsel[g], 0, 0)
    cw1 = lambda g, tix, wsel, init, r, wb, *s: (wb[g], 0)
    cw2 = lambda g, tix, wsel, init, r, wb, *s: (wb[g] + 1, 0)
    pw1 = lambda g, tix, wsel, init, r, wb, wbase, pwb, *s: (pwb[g], 0)
    pw2 = lambda g, tix, wsel, init, r, wb, wbase, pwb, *s: (pwb[g] + 1, 0)
    grid_spec = pltpu.PrefetchScalarGridSpec(
        num_scalar_prefetch=9,
        grid=(nsteps,),
        in_specs=[
            pl.BlockSpec((TR, D_MODEL), rd),
            pl.BlockSpec((TR, 1), rd),
            pl.BlockSpec((1, 1, TR), rd3),
            pl.BlockSpec((W, D_FFN), cw1),
            pl.BlockSpec((W, D_FFN), cw2),
            pl.BlockSpec((1, D_MODEL, D_FFN), wt),
            pl.BlockSpec((1, D_MODEL, D_FFN), wt),
            pl.BlockSpec((1, D_FFN, D_MODEL), wt),
            _fullp((D_MODEL, 1024)), _fullp((1, 1024)),
            _fullp((1024, 1024)), _fullp((1, 1024)),
        ],
        out_specs=[
            pl.BlockSpec((W, 1024), pw1),
            pl.BlockSpec((W, 1024), pw2),
        ],
    )
    return pl.pallas_call(
        body,
        grid_spec=grid_spec,
        out_shape=[jax.ShapeDtypeStruct((V + W, 1024), jnp.float32),
                   jax.ShapeDtypeStruct((V + W, 1024), jnp.float32)],
    )(*scalars, x, seg_col, seg_row, ctx_tab, ctx_tab, W1b, W2b, W3b,
      W_p1, b_p1, W_p2, b_p2)


def _sc_gather(table, idx, padT):
    from jax.experimental.pallas import tpu_sc as plsc
    info = plsc.get_sparse_core_info()
    NC, NS = info.num_cores, info.num_subcores
    NW = NC * NS
    CH = 128
    b_per_w = padT // NW
    nch = b_per_w // CH
    mesh = plsc.VectorSubcoreMesh(core_axis_name="c", subcore_axis_name="s")

    @functools.partial(
        pl.kernel, mesh=mesh,
        out_type=jax.ShapeDtypeStruct((padT, 256), jnp.float32),
        scratch_types=[
            pltpu.VMEM((CH,), jnp.int32),
            pltpu.VMEM((CH, 256), jnp.float32),
            pltpu.SemaphoreType.DMA,
        ],
    )
    def k(table_hbm, idx_hbm, out_hbm, idx_v, rows_v, sem):
        wid = lax.axis_index("s") * NC + lax.axis_index("c")
        base = wid * b_per_w
        for j in range(nch):
            off = base + j * CH
            pltpu.sync_copy(idx_hbm.at[pl.ds(off, CH)], idx_v)
            pltpu.async_copy(table_hbm.at[idx_v], rows_v, sem).wait()
            pltpu.sync_copy(rows_v, out_hbm.at[pl.ds(off, CH)])

    return k(table, idx)


def _final_body(pooled_ref, ac_ref, wf1, bf1, wf2, bf2, out_ref):
    pooled = pooled_ref[...] / ac_ref[...]
    h = jnp.maximum(_bdot(pooled, wf1[...]) + bf1[...], 0.0)
    out_ref[...] = _bdot(h, wf2[...]) + bf2[...]


def _final_stage(pool_sum, ac, W_f1, b_f1, W_f2, b_f2):
    return pl.pallas_call(
        _final_body,
        grid=(V // BV,),
        in_specs=[
            pl.BlockSpec((BV, 1024), lambda i: (i, 0)),
            pl.BlockSpec((BV, 1), lambda i: (i, 0)),
            _full((1024, 512)), _full((1, 512)),
            _full((512, 256)), _full((1, 256)),
        ],
        out_specs=pl.BlockSpec((BV, 256), lambda i: (i, 0)),
        out_shape=jax.ShapeDtypeStruct((V, 256), jnp.float32),
    )(pool_sum, ac, W_f1, b_f1, W_f2, b_f2)


def _merge_windows(o1, o2, n):
    return o1[:n] + jnp.concatenate(
        [jnp.zeros((W, o2.shape[1]), o2.dtype), o2[W:n]], axis=0)


def kernel(reads_2d, info_2d, ref_seq_2d, W_r1, b_r1, W_r2, b_r2, W_i1, b_i1,
           W_i2, b_i2, W_conv, b_conv, W_seq, b_seq, enc_W1, enc_W2, enc_Wc,
           enc_W3, W_p1, b_p1, W_p2, b_p2, W_f1, b_f1, W_f2, b_f2,
           ref_counts, alt_counts):
    total = reads_2d.shape[0]
    ntiles = (total + TR - 1) // TR

    counts2 = jnp.concatenate((ref_counts, alt_counts)).astype(jnp.int32)
    seg2 = jnp.repeat(jnp.arange(2 * V, dtype=jnp.int32), counts2,
                      total_repeat_length=total)
    is_alt = seg2 >= V
    var_all = jnp.where(is_alt, seg2 - V, seg2)
    R = jnp.sum(ref_counts).astype(jnp.int32)

    rc = ref_counts.astype(jnp.float32).reshape(V, 1)
    ac = alt_counts.astype(jnp.float32).reshape(V, 1)

    seg_pad = jnp.concatenate(
        [seg2, jnp.full((ntiles * TR - total,), BIG, jnp.int32)])
    seg_col = seg_pad.reshape(ntiles * TR, 1)
    seg_row = seg_pad.reshape(ntiles, 1, TR)

    seg_start = seg2[::TR]
    wb_t = (seg_start // W).astype(jnp.int32)
    va_t = jnp.maximum(seg_start - V, 0)
    pwb_t = (va_t // W).astype(jnp.int32)

    ir = jnp.concatenate([jnp.ones((1,), jnp.int32),
                          (wb_t[1:] != wb_t[:-1]).astype(jnp.int32)])

    nsteps = ntiles + 1
    s_t = R // TR
    gidx = jnp.arange(nsteps, dtype=jnp.int32)
    tix = jnp.where(gidx <= s_t, gidx, gidx - 1).astype(jnp.int32)
    wsel = (gidx > s_t).astype(jnp.int32)
    init = jnp.where(gidx == s_t + 1, 0, 1).astype(jnp.int32)
    Rarr = jnp.stack([R, jnp.full((), V, jnp.int32)]).astype(jnp.int32)
    wb_g = wb_t[tix]
    wbase_g = wb_g * W
    sinit = jnp.concatenate([jnp.ones((1,), jnp.int32),
                             (wb_g[1:] != wb_g[:-1]).astype(jnp.int32)])
    pwb_g = pwb_t[tix]
    pwbase_g = pwb_g * W + V
    pinit = jnp.concatenate([jnp.ones((1,), jnp.int32),
                             (pwb_g[1:] != pwb_g[:-1]).astype(jnp.int32)])

    x3 = ref_seq_2d.reshape(V, 4, 64)
    cols = [x3[:, i, k:k + 60] for i in range(4) for k in range(5)]
    patches = jnp.stack(cols + [jnp.zeros((V, 60), jnp.float32)] * 4, axis=-1)
    patches = patches.reshape(V * 60, 24)
    W_conv2d = jnp.concatenate(
        [W_conv.reshape(64, 20).T, jnp.zeros((4, 64), jnp.float32)], axis=0)

    iseq = _variant_stage(info_2d, patches, W_i1, b_i1.reshape(1, -1),
                          W_i2, b_i2.reshape(1, -1), W_conv2d,
                          b_conv.reshape(1, -1), W_seq, b_seq.reshape(1, -1))
    ref_seq_embeddings_ve = iseq[:, 128:]

    xa, o1, o2 = _read_stage(wb_t, wb_t * W, ir, reads_2d, seg_row,
                             W_r1, b_r1.reshape(1, -1), W_r2,
                             b_r2.reshape(1, -1), ntiles, total)
    sum_a = _merge_windows(o1, o2, 2 * V)

    padT = ((ntiles * TR + 4095) // 4096) * 4096
    idx_pad = jnp.concatenate(
        [var_all, jnp.zeros((padT - total,), jnp.int32)]).astype(jnp.int32)
    iseq_g = _sc_gather(iseq, idx_pad, padT)[:ntiles * TR]

    cnt_iseq = jnp.tile(iseq, (2, 1)) * counts2[:, None].astype(jnp.float32)
    sums = jnp.concatenate([sum_a, cnt_iseq], axis=1)

    zpad = jnp.zeros((W, 1024), jnp.float32)
    ctx0, ctx1 = _ctx_stage(sums[:V], sums[V:], rc, ac,
                            enc_Wc[0, 0], enc_Wc[0, 1])
    ctx_tab = jnp.concatenate([ctx0, ctx1, zpad], axis=0)

    scalars0 = (tix, wsel, init, Rarr, wb_g, wbase_g, sinit)
    x1, d1, d2 = _enc0_stage(scalars0, xa, iseq_g, seg_col, seg_row, ctx_tab,
                             enc_W1[0], enc_W2[0], enc_W3[0], nsteps, total)
    sums1 = sums + _merge_windows(d1, d2, 2 * V)

    ctx0b, ctx1b = _ctx_stage(sums1[:V], sums1[V:], rc, ac,
                              enc_Wc[1, 0], enc_Wc[1, 1])
    ctx_tab1 = jnp.concatenate([ctx0b, ctx1b, zpad], axis=0)

    scalars1 = (tix, wsel, init, Rarr, wb_g, wbase_g, pwb_g, pwbase_g, pinit)
    p1, p2 = _enc1_stage(scalars1, x1, seg_col, seg_row, ctx_tab1,
                         enc_W1[1], enc_W2[1], enc_W3[1],
                         W_p1, b_p1.reshape(1, -1), W_p2, b_p2.reshape(1, -1),
                         nsteps, total)
    pool_sum = _merge_windows(p1, p2, V)

    result_be = _final_stage(pool_sum, ac, W_f1, b_f1.reshape(1, -1),
                             W_f2, b_f2.reshape(1, -1))
    return result_be, ref_seq_embeddings_ve
